# Initial kernel scaffold; baseline (speedup 1.0000x reference)
import jax
import jax.numpy as jnp
from jax.experimental import pallas as pl


def kernel(edge_index, etype, norm, in_edges_mask, out_edges_mask, sub, rel, n_embds, rel_embds, loop_rel, W_O, W_I, W_S, W_R, conv_w, fc_w, fc_b):
    raise NotImplementedError("write your pallas kernel here")


# speedup vs baseline: 1.4889x; 1.4889x over previous
import functools

import numpy as np
import jax
import jax.numpy as jnp
from jax import lax
from jax.experimental import pallas as pl
from jax.experimental.pallas import tpu as pltpu
from jax.experimental.pallas import tpu_sc as plsc

N_ENT = 10000
N_REL = 64
D = 128
E = 320000
B = 1024
KER = 7
NFILT = 96
OW = 10
OH = 10
FLAT = NFILT * OW * OH

NC = 2
NS = 16
EP = 327680
ROWS_PER_SUB = EP // 128 // NS
ROWS_OUT = 624

def _vbroadcast(vec, e):
    idx = jnp.full((16, 1), e, jnp.int32)
    dnums = lax.GatherDimensionNumbers(
        offset_dims=(), collapsed_slice_dims=(0,), start_index_map=(0,))
    return lax.gather(vec, idx, dnums, (1,),
                      mode=lax.GatherScatterMode.PROMISE_IN_BOUNDS)


_sc_mesh = plsc.VectorSubcoreMesh(core_axis_name="c", subcore_axis_name="s")


@functools.partial(
    pl.kernel,
    mesh=_sc_mesh,
    compiler_params=pltpu.CompilerParams(needs_layout_passes=False),
    out_type=jax.ShapeDtypeStruct((NC, N_ENT, D), jnp.float32),
    scratch_types=[
        pltpu.VMEM((1, 128), jnp.int32),
        pltpu.VMEM((1, 128), jnp.int32),
        pltpu.VMEM((1, 128), jnp.int32),
        pltpu.VMEM((1, 128), jnp.float32),
        pltpu.VMEM((1, 128), jnp.int32),
        pltpu.VMEM((128, D), jnp.float32),
        pltpu.VMEM((128, D), jnp.float32),
        pltpu.VMEM_SHARED((N_ENT, D), jnp.float32),
        pltpu.SemaphoreType.DMA,
    ],
)
def _sc_edge(src_hbm, dst_hbm, et_hbm, nm_hbm, mc_hbm, ntab_hbm, rtab_hbm,
             out_hbm, src_v, dst_v, et_v, nm_v, mc_v, rows_v, rrows_v,
             acc_sh, sem):
    c = lax.axis_index("c")
    s = lax.axis_index("s")

    def _zero_row(i, _):
        for j in range(D // 16):
            rows_v[i, pl.ds(j * 16, 16)] = jnp.zeros((16,), jnp.float32)
        return 0

    lax.fori_loop(0, 128, _zero_row, 0)
    for t in range(ROWS_OUT // 128):
        pltpu.sync_copy(rows_v, acc_sh.at[pl.ds(s * ROWS_OUT + t * 128, 128)])
    pltpu.sync_copy(rows_v.at[pl.ds(0, ROWS_OUT - 512)],
                    acc_sh.at[pl.ds(s * ROWS_OUT + 512, ROWS_OUT - 512)])

    @pl.when(s == NS - 1)
    def _zero_tail():
        pltpu.sync_copy(rows_v.at[pl.ds(0, N_ENT - NS * ROWS_OUT)],
                        acc_sh.at[pl.ds(NS * ROWS_OUT, N_ENT - NS * ROWS_OUT)])

    plsc.subcore_barrier()

    def _block(blk, _):
        r0 = s * ROWS_PER_SUB + blk
        pltpu.sync_copy(src_hbm.at[pl.ds(r0, 1)], src_v)
        pltpu.sync_copy(dst_hbm.at[pl.ds(r0, 1)], dst_v)
        pltpu.sync_copy(et_hbm.at[pl.ds(r0, 1)], et_v)
        pltpu.sync_copy(nm_hbm.at[pl.ds(r0, 1)], nm_v)
        pltpu.sync_copy(mc_hbm.at[pl.ds(r0, 1)], mc_v)
        cpn = pltpu.async_copy(ntab_hbm.at[src_v.at[0]], rows_v, sem)
        cpr = pltpu.async_copy(rtab_hbm.at[et_v.at[0]], rrows_v, sem)
        cpn.wait()
        cpr.wait()

        def _group(g, _):
            l0 = g * 16
            nvec = nm_v[0, pl.ds(l0, 16)]
            mvec = mc_v[0, pl.ds(l0, 16)]
            scale = jnp.where(mvec == c, nvec, 0.0)
            for e in range(16):
                sspl = _vbroadcast(scale, e)
                row = l0 + e
                for j in range(D // 16):
                    rows_v[row, pl.ds(j * 16, 16)] = (
                        rows_v[row, pl.ds(j * 16, 16)]
                        * rrows_v[row, pl.ds(j * 16, 16)] * sspl)
            return 0

        lax.fori_loop(0, 8, _group, 0)
        pltpu.sync_copy(rows_v, acc_sh.at[dst_v.at[0]], add=True)
        return 0

    lax.fori_loop(0, ROWS_PER_SUB, _block, 0)
    plsc.subcore_barrier()
    pltpu.sync_copy(acc_sh.at[pl.ds(s * ROWS_OUT, ROWS_OUT)],
                    out_hbm.at[c, pl.ds(s * ROWS_OUT, ROWS_OUT)])

    @pl.when(s == NS - 1)
    def _out_tail():
        pltpu.sync_copy(acc_sh.at[pl.ds(NS * ROWS_OUT, N_ENT - NS * ROWS_OUT)],
                        out_hbm.at[c, pl.ds(NS * ROWS_OUT,
                                            N_ENT - NS * ROWS_OUT)])


def _mm_t(a, b):
    return lax.dot_general(a, b, (((1,), (1,)), ((), ())),
                           preferred_element_type=jnp.float32)


def _k1_body(acc_ref, nemb_ref, remb_ref, loop_ref, wi_ref, wo_ref, ws_ref,
             wr_ref, sub_ref, rel_ref, nf_ref, img_ref):
    comp_edge = _mm_t(acc_ref[0], wi_ref[...]) + _mm_t(acc_ref[1], wo_ref[...])
    pre = (_mm_t(nemb_ref[...] * loop_ref[...][0][None, :], ws_ref[...])
           + comp_edge) * (1.0 / 3.0)
    m = jnp.mean(pre, axis=0)
    cen = pre - m[None, :]
    v = jnp.mean(cen * cen, axis=0)
    nf = jnp.tanh(cen * lax.rsqrt(v + 1e-5)[None, :])
    nf_ref[...] = nf
    r_feats = _mm_t(remb_ref[...], wr_ref[...])
    sub_c = sub_ref[...]
    CH = 1000
    acc = jnp.zeros((B, D), jnp.float32)
    for k in range(N_ENT // CH):
        it = lax.broadcasted_iota(jnp.int32, (1, CH), 1) + (k * CH)
        oh = jnp.where(sub_c == it, 1.0, 0.0)
        acc = acc + jnp.dot(oh, nf[k * CH:(k + 1) * CH],
                            preferred_element_type=jnp.float32)
    sub_e = acc
    rel_c = rel_ref[...]
    itr = lax.broadcasted_iota(jnp.int32, (1, N_REL), 1)
    ohr = jnp.where(rel_c == itr, 1.0, 0.0)
    rel_e = jnp.dot(ohr, r_feats, preferred_element_type=jnp.float32)
    u_i = lax.broadcasted_iota(jnp.int32, (D, 2 * D), 1)
    d_i = lax.broadcasted_iota(jnp.int32, (D, 2 * D), 0)
    pe = jnp.where(u_i == 2 * d_i, 1.0, 0.0)
    po = jnp.where(u_i == 2 * d_i + 1, 1.0, 0.0)
    img = (jnp.dot(sub_e, pe, preferred_element_type=jnp.float32)
           + jnp.dot(rel_e, po, preferred_element_type=jnp.float32))
    m0 = jnp.mean(img)
    cen0 = img - m0
    v0 = jnp.mean(cen0 * cen0)
    img_ref[...] = cen0 * lax.rsqrt(v0 + 1e-5)


K2_GRID = 3
FPG = NFILT // K2_GRID
CW = FPG * OW * OH


def _k2_body(img_ref, t2_ref, fcw_ref, fcb_ref, x_ref, acc_ref):
    g = pl.program_id(0)
    y = jnp.dot(img_ref[...], t2_ref[...], preferred_element_type=jnp.float32)
    f_i = lax.broadcasted_iota(jnp.int32, (FPG, CW), 0)
    c_i = lax.broadcasted_iota(jnp.int32, (FPG, CW), 1)
    R = jnp.where(f_i == c_i // (OW * OH), 1.0, 0.0)
    colmean = jnp.mean(y, axis=0)[None, :]
    m1 = _mm_t(colmean, R) * (1.0 / (OW * OH))
    m1c = jnp.dot(m1, R, preferred_element_type=jnp.float32)
    yc = y - m1c
    colsq = jnp.mean(yc * yc, axis=0)[None, :]
    v1 = _mm_t(colsq, R) * (1.0 / (OW * OH))
    s1c = jnp.dot(lax.rsqrt(v1 + 1e-5), R, preferred_element_type=jnp.float32)
    y = jnp.maximum(yc * s1c, 0.0)
    part = _mm_t(y, fcw_ref[...])

    @pl.when(g == 0)
    def _init():
        acc_ref[...] = part

    @pl.when(g > 0)
    def _accum():
        acc_ref[...] = acc_ref[...] + part

    @pl.when(g == K2_GRID - 1)
    def _finish():
        x = acc_ref[...] + fcb_ref[...]
        m2 = jnp.mean(x, axis=0)
        cen2 = x - m2[None, :]
        v2 = jnp.mean(cen2 * cen2, axis=0)
        x_ref[...] = jnp.maximum(cen2 * lax.rsqrt(v2 + 1e-5)[None, :], 0.0)


def _k3_body(x_ref, nf_ref, out_ref):
    out_ref[...] = _mm_t(x_ref[...], nf_ref[...])


def _build_sel() -> np.ndarray:
    s = np.zeros((KER * KER, 2 * D, OW * OH), np.float32)
    for p in range(OW):
        for q in range(OH):
            for i in range(KER):
                for j in range(KER):
                    s[i * KER + j, (p + i) * 16 + (q + j), p * OH + q] = 1.0
    return s.reshape(KER * KER, 2 * D * OW * OH)


_SEL = _build_sel()


def _conv_toeplitz(conv_w):
    w49 = conv_w[:, 0].reshape(NFILT, KER * KER)
    t = jnp.dot(w49, jnp.asarray(_SEL))
    return t.reshape(NFILT, 2 * D, OW * OH).transpose(1, 0, 2).reshape(
        2 * D, FLAT)


def kernel(edge_index, etype, norm, in_edges_mask, out_edges_mask, sub, rel,
           n_embds, rel_embds, loop_rel, W_O, W_I, W_S, W_R, conv_w, fc_w,
           fc_b):
    src = edge_index[0]
    dst = edge_index[1]
    mcode = out_edges_mask.astype(jnp.int32)
    nflat = norm[:, 0]
    pad = EP - E
    src_p = jnp.pad(src, (0, pad)).reshape(EP // 128, 128)
    dst_p = jnp.pad(dst, (0, pad)).reshape(EP // 128, 128)
    et_p = jnp.pad(etype, (0, pad)).reshape(EP // 128, 128)
    nm_p = jnp.pad(nflat, (0, pad)).reshape(EP // 128, 128)
    mc_p = jnp.pad(mcode, (0, pad)).reshape(EP // 128, 128)
    rtab = jnp.concatenate([rel_embds, loop_rel], axis=0)

    acc = _sc_edge(src_p, dst_p, et_p, nm_p, mc_p, n_embds, rtab)

    nf, img = pl.pallas_call(
        _k1_body,
        out_shape=(
            jax.ShapeDtypeStruct((N_ENT, D), jnp.float32),
            jax.ShapeDtypeStruct((B, 2 * D), jnp.float32),
        ),
    )(acc, n_embds, rel_embds, loop_rel, W_I, W_O, W_S, W_R,
      sub.reshape(B, 1), rel.reshape(B, 1))

    t2 = _conv_toeplitz(conv_w)
    x = pl.pallas_call(
        _k2_body,
        grid=(K2_GRID,),
        in_specs=[
            pl.BlockSpec((B, 2 * D), lambda g: (0, 0)),
            pl.BlockSpec((2 * D, CW), lambda g: (0, g)),
            pl.BlockSpec((D, CW), lambda g: (0, g)),
            pl.BlockSpec((1, D), lambda g: (0, 0)),
        ],
        out_specs=pl.BlockSpec((B, D), lambda g: (0, 0)),
        scratch_shapes=[pltpu.VMEM((B, D), jnp.float32)],
        out_shape=jax.ShapeDtypeStruct((B, D), jnp.float32),
    )(img, t2, fc_w, fc_b.reshape(1, D))

    scores = pl.pallas_call(
        _k3_body,
        out_shape=jax.ShapeDtypeStruct((B, N_ENT), jnp.float32),
    )(x, nf)
    return scores


# speedup vs baseline: 1.8045x; 1.2119x over previous
import functools

import numpy as np
import jax
import jax.numpy as jnp
from jax import lax
from jax.experimental import pallas as pl
from jax.experimental.pallas import tpu as pltpu
from jax.experimental.pallas import tpu_sc as plsc

N_ENT = 10000
N_REL = 64
D = 128
E = 320000
B = 1024
KER = 7
NFILT = 96
OW = 10
OH = 10
FLAT = NFILT * OW * OH

NC = 2
NS = 16
EP = 327680
ROWS_PER_SUB = EP // 128 // NS
BLOCKS = ROWS_PER_SUB // 2
ROWS_OUT = 624

def _vbroadcast(vec, e):
    idx = jnp.full((16, 1), e, jnp.int32)
    dnums = lax.GatherDimensionNumbers(
        offset_dims=(), collapsed_slice_dims=(0,), start_index_map=(0,))
    return lax.gather(vec, idx, dnums, (1,),
                      mode=lax.GatherScatterMode.PROMISE_IN_BOUNDS)


_sc_mesh = plsc.VectorSubcoreMesh(core_axis_name="c", subcore_axis_name="s")


@functools.partial(
    pl.kernel,
    mesh=_sc_mesh,
    compiler_params=pltpu.CompilerParams(needs_layout_passes=False),
    out_type=jax.ShapeDtypeStruct((NC, N_ENT, D), jnp.float32),
    scratch_types=[
        pltpu.VMEM((2, 128), jnp.int32),
        pltpu.VMEM((2, 128), jnp.int32),
        pltpu.VMEM((2, 128), jnp.int32),
        pltpu.VMEM((2, 128), jnp.float32),
        pltpu.VMEM((2, 128), jnp.int32),
        pltpu.VMEM((256, D), jnp.float32),
        pltpu.VMEM(((N_REL + 1) * D,), jnp.float32),
        pltpu.VMEM_SHARED((N_ENT, D), jnp.float32),
        pltpu.SemaphoreType.DMA,
        pltpu.SemaphoreType.DMA,
    ],
)
def _sc_edge(src_hbm, dst_hbm, et_hbm, nm_hbm, mc_hbm, ntab_hbm, rtab_hbm,
             out_hbm, src_v, dst_v, et_v, nm_v, mc_v, rows_v, rtab_v,
             acc_sh, sem, sem2):
    c = lax.axis_index("c")
    s = lax.axis_index("s")

    pltpu.sync_copy(rtab_hbm, rtab_v)

    def _zero_row(i, _):
        for j in range(D // 16):
            rows_v[i, pl.ds(j * 16, 16)] = jnp.zeros((16,), jnp.float32)
        return 0

    lax.fori_loop(0, 256, _zero_row, 0)
    pltpu.sync_copy(rows_v, acc_sh.at[pl.ds(s * ROWS_OUT, 256)])
    pltpu.sync_copy(rows_v, acc_sh.at[pl.ds(s * ROWS_OUT + 256, 256)])
    pltpu.sync_copy(rows_v.at[pl.ds(0, ROWS_OUT - 512)],
                    acc_sh.at[pl.ds(s * ROWS_OUT + 512, ROWS_OUT - 512)])

    @pl.when(s == NS - 1)
    def _zero_tail():
        pltpu.sync_copy(rows_v.at[pl.ds(0, N_ENT - NS * ROWS_OUT)],
                        acc_sh.at[pl.ds(NS * ROWS_OUT, N_ENT - NS * ROWS_OUT)])

    plsc.subcore_barrier()

    def _block(blk, _):
        r0 = s * ROWS_PER_SUB + blk * 2
        pltpu.sync_copy(src_hbm.at[pl.ds(r0, 2)], src_v)
        pltpu.sync_copy(dst_hbm.at[pl.ds(r0, 2)], dst_v)
        pltpu.sync_copy(et_hbm.at[pl.ds(r0, 2)], et_v)
        pltpu.sync_copy(nm_hbm.at[pl.ds(r0, 2)], nm_v)
        pltpu.sync_copy(mc_hbm.at[pl.ds(r0, 2)], mc_v)
        cps = [pltpu.async_copy(ntab_hbm.at[src_v.at[k]],
                                rows_v.at[pl.ds(k * 128, 128)], sm)
               for k, sm in ((0, sem), (1, sem2))]

        def _group(g, _):
            k = g >> 3
            l0 = (g & 7) * 16
            tvec = et_v[k, pl.ds(l0, 16)]
            nvec = nm_v[k, pl.ds(l0, 16)]
            mvec = mc_v[k, pl.ds(l0, 16)]
            scale = jnp.where(mvec == c, nvec, 0.0)
            for e in range(16):
                tspl = _vbroadcast(tvec, e)
                sspl = _vbroadcast(scale, e)
                base = tspl * D + lax.iota(jnp.int32, 16)
                row = g * 16 + e
                for j in range(D // 16):
                    rr = plsc.load_gather(rtab_v, [base + j * 16])
                    rows_v[row, pl.ds(j * 16, 16)] = (
                        rows_v[row, pl.ds(j * 16, 16)] * rr * sspl)
            return 0

        for k in range(2):
            cps[k].wait()
            lax.fori_loop(8 * k, 8 * (k + 1), _group, 0)
            pltpu.sync_copy(rows_v.at[pl.ds(k * 128, 128)],
                            acc_sh.at[dst_v.at[k]], add=True)
        return 0

    lax.fori_loop(0, BLOCKS, _block, 0)
    plsc.subcore_barrier()
    pltpu.sync_copy(acc_sh.at[pl.ds(s * ROWS_OUT, ROWS_OUT)],
                    out_hbm.at[c, pl.ds(s * ROWS_OUT, ROWS_OUT)])

    @pl.when(s == NS - 1)
    def _out_tail():
        pltpu.sync_copy(acc_sh.at[pl.ds(NS * ROWS_OUT, N_ENT - NS * ROWS_OUT)],
                        out_hbm.at[c, pl.ds(NS * ROWS_OUT,
                                            N_ENT - NS * ROWS_OUT)])


def _mm_t(a, b):
    return lax.dot_general(a, b, (((1,), (1,)), ((), ())),
                           preferred_element_type=jnp.float32)


def _k1_body(acc_ref, nemb_ref, remb_ref, loop_ref, wi_ref, wo_ref, ws_ref,
             wr_ref, sub_ref, rel_ref, nf_ref, img_ref):
    comp_edge = _mm_t(acc_ref[0], wi_ref[...]) + _mm_t(acc_ref[1], wo_ref[...])
    pre = (_mm_t(nemb_ref[...] * loop_ref[...][0][None, :], ws_ref[...])
           + comp_edge) * (1.0 / 3.0)
    m = jnp.mean(pre, axis=0)
    cen = pre - m[None, :]
    v = jnp.mean(cen * cen, axis=0)
    nf = jnp.tanh(cen * lax.rsqrt(v + 1e-5)[None, :])
    nf_ref[...] = nf
    r_feats = _mm_t(remb_ref[...], wr_ref[...])
    sub_c = sub_ref[...]
    CH = 1000
    acc = jnp.zeros((B, D), jnp.float32)
    for k in range(N_ENT // CH):
        it = lax.broadcasted_iota(jnp.int32, (1, CH), 1) + (k * CH)
        oh = jnp.where(sub_c == it, 1.0, 0.0)
        acc = acc + jnp.dot(oh, nf[k * CH:(k + 1) * CH],
                            preferred_element_type=jnp.float32)
    sub_e = acc
    rel_c = rel_ref[...]
    itr = lax.broadcasted_iota(jnp.int32, (1, N_REL), 1)
    ohr = jnp.where(rel_c == itr, 1.0, 0.0)
    rel_e = jnp.dot(ohr, r_feats, preferred_element_type=jnp.float32)
    u_i = lax.broadcasted_iota(jnp.int32, (D, 2 * D), 1)
    d_i = lax.broadcasted_iota(jnp.int32, (D, 2 * D), 0)
    pe = jnp.where(u_i == 2 * d_i, 1.0, 0.0)
    po = jnp.where(u_i == 2 * d_i + 1, 1.0, 0.0)
    img = (jnp.dot(sub_e, pe, preferred_element_type=jnp.float32)
           + jnp.dot(rel_e, po, preferred_element_type=jnp.float32))
    m0 = jnp.mean(img)
    cen0 = img - m0
    v0 = jnp.mean(cen0 * cen0)
    img_ref[...] = cen0 * lax.rsqrt(v0 + 1e-5)


K2_GRID = 3
FPG = NFILT // K2_GRID
CW = FPG * OW * OH


def _k2_body(img_ref, t2_ref, fcw_ref, fcb_ref, x_ref, acc_ref):
    g = pl.program_id(0)
    y = jnp.dot(img_ref[...], t2_ref[...], preferred_element_type=jnp.float32)
    f_i = lax.broadcasted_iota(jnp.int32, (FPG, CW), 0)
    c_i = lax.broadcasted_iota(jnp.int32, (FPG, CW), 1)
    R = jnp.where(f_i == c_i // (OW * OH), 1.0, 0.0)
    colmean = jnp.mean(y, axis=0)[None, :]
    m1 = _mm_t(colmean, R) * (1.0 / (OW * OH))
    m1c = jnp.dot(m1, R, preferred_element_type=jnp.float32)
    yc = y - m1c
    colsq = jnp.mean(yc * yc, axis=0)[None, :]
    v1 = _mm_t(colsq, R) * (1.0 / (OW * OH))
    s1c = jnp.dot(lax.rsqrt(v1 + 1e-5), R, preferred_element_type=jnp.float32)
    y = jnp.maximum(yc * s1c, 0.0)
    part = _mm_t(y, fcw_ref[...])

    @pl.when(g == 0)
    def _init():
        acc_ref[...] = part

    @pl.when(g > 0)
    def _accum():
        acc_ref[...] = acc_ref[...] + part

    @pl.when(g == K2_GRID - 1)
    def _finish():
        x = acc_ref[...] + fcb_ref[...]
        m2 = jnp.mean(x, axis=0)
        cen2 = x - m2[None, :]
        v2 = jnp.mean(cen2 * cen2, axis=0)
        x_ref[...] = jnp.maximum(cen2 * lax.rsqrt(v2 + 1e-5)[None, :], 0.0)


def _k3_body(x_ref, nf_ref, out_ref):
    out_ref[...] = _mm_t(x_ref[...], nf_ref[...])


def _build_sel() -> np.ndarray:
    s = np.zeros((KER * KER, 2 * D, OW * OH), np.float32)
    for p in range(OW):
        for q in range(OH):
            for i in range(KER):
                for j in range(KER):
                    s[i * KER + j, (p + i) * 16 + (q + j), p * OH + q] = 1.0
    return s.reshape(KER * KER, 2 * D * OW * OH)


_SEL = _build_sel()


def _conv_toeplitz(conv_w):
    w49 = conv_w[:, 0].reshape(NFILT, KER * KER)
    t = jnp.dot(w49, jnp.asarray(_SEL))
    return t.reshape(NFILT, 2 * D, OW * OH).transpose(1, 0, 2).reshape(
        2 * D, FLAT)


def kernel(edge_index, etype, norm, in_edges_mask, out_edges_mask, sub, rel,
           n_embds, rel_embds, loop_rel, W_O, W_I, W_S, W_R, conv_w, fc_w,
           fc_b):
    src = edge_index[0]
    dst = edge_index[1]
    mcode = out_edges_mask.astype(jnp.int32)
    nflat = norm[:, 0]
    pad = EP - E
    src_p = jnp.pad(src, (0, pad)).reshape(EP // 128, 128)
    dst_p = jnp.pad(dst, (0, pad)).reshape(EP // 128, 128)
    et_p = jnp.pad(etype, (0, pad)).reshape(EP // 128, 128)
    nm_p = jnp.pad(nflat, (0, pad)).reshape(EP // 128, 128)
    mc_p = jnp.pad(mcode, (0, pad)).reshape(EP // 128, 128)
    rtab = jnp.concatenate([rel_embds, loop_rel], axis=0).reshape(-1)

    acc = _sc_edge(src_p, dst_p, et_p, nm_p, mc_p, n_embds, rtab)

    nf, img = pl.pallas_call(
        _k1_body,
        out_shape=(
            jax.ShapeDtypeStruct((N_ENT, D), jnp.float32),
            jax.ShapeDtypeStruct((B, 2 * D), jnp.float32),
        ),
    )(acc, n_embds, rel_embds, loop_rel, W_I, W_O, W_S, W_R,
      sub.reshape(B, 1), rel.reshape(B, 1))

    t2 = _conv_toeplitz(conv_w)
    x = pl.pallas_call(
        _k2_body,
        grid=(K2_GRID,),
        in_specs=[
            pl.BlockSpec((B, 2 * D), lambda g: (0, 0)),
            pl.BlockSpec((2 * D, CW), lambda g: (0, g)),
            pl.BlockSpec((D, CW), lambda g: (0, g)),
            pl.BlockSpec((1, D), lambda g: (0, 0)),
        ],
        out_specs=pl.BlockSpec((B, D), lambda g: (0, 0)),
        scratch_shapes=[pltpu.VMEM((B, D), jnp.float32)],
        out_shape=jax.ShapeDtypeStruct((B, D), jnp.float32),
    )(img, t2, fc_w, fc_b.reshape(1, D))

    scores = pl.pallas_call(
        _k3_body,
        out_shape=jax.ShapeDtypeStruct((B, N_ENT), jnp.float32),
    )(x, nf)
    return scores


# speedup vs baseline: 1.8986x; 1.0522x over previous
import functools

import numpy as np
import jax
import jax.numpy as jnp
from jax import lax
from jax.experimental import pallas as pl
from jax.experimental.pallas import tpu as pltpu
from jax.experimental.pallas import tpu_sc as plsc

N_ENT = 10000
N_REL = 64
D = 128
E = 320000
B = 1024
KER = 7
NFILT = 96
OW = 10
OH = 10
FLAT = NFILT * OW * OH

NC = 2
NS = 16
EP = 327680
ROWS_PER_SUB = EP // 128 // NS
BLOCKS = ROWS_PER_SUB // 2
ROWS_OUT = 624

def _vbroadcast(vec, e):
    idx = jnp.full((16, 1), e, jnp.int32)
    dnums = lax.GatherDimensionNumbers(
        offset_dims=(), collapsed_slice_dims=(0,), start_index_map=(0,))
    return lax.gather(vec, idx, dnums, (1,),
                      mode=lax.GatherScatterMode.PROMISE_IN_BOUNDS)


_sc_mesh = plsc.VectorSubcoreMesh(core_axis_name="c", subcore_axis_name="s")


@functools.partial(
    pl.kernel,
    mesh=_sc_mesh,
    compiler_params=pltpu.CompilerParams(needs_layout_passes=False),
    out_type=jax.ShapeDtypeStruct((NC, N_ENT, D), jnp.float32),
    scratch_types=[
        pltpu.VMEM((16, 128), jnp.int32),
        pltpu.VMEM((256, D), jnp.float32),
        pltpu.VMEM(((N_REL + 1) * D,), jnp.float32),
        pltpu.VMEM_SHARED((N_ENT, D), jnp.float32),
        pltpu.SemaphoreType.DMA,
        pltpu.SemaphoreType.DMA,
    ],
)
def _sc_edge(meta_hbm, ntab_hbm, rtab_hbm,
             out_hbm, meta_v, rows_v, rtab_v,
             acc_sh, sem, sem2):
    c = lax.axis_index("c")
    s = lax.axis_index("s")

    pltpu.sync_copy(rtab_hbm, rtab_v)

    def _zero_row(i, _):
        for j in range(D // 16):
            rows_v[i, pl.ds(j * 16, 16)] = jnp.zeros((16,), jnp.float32)
        return 0

    lax.fori_loop(0, 256, _zero_row, 0)
    pltpu.sync_copy(rows_v, acc_sh.at[pl.ds(s * ROWS_OUT, 256)])
    pltpu.sync_copy(rows_v, acc_sh.at[pl.ds(s * ROWS_OUT + 256, 256)])
    pltpu.sync_copy(rows_v.at[pl.ds(0, ROWS_OUT - 512)],
                    acc_sh.at[pl.ds(s * ROWS_OUT + 512, ROWS_OUT - 512)])

    @pl.when(s == NS - 1)
    def _zero_tail():
        pltpu.sync_copy(rows_v.at[pl.ds(0, N_ENT - NS * ROWS_OUT)],
                        acc_sh.at[pl.ds(NS * ROWS_OUT, N_ENT - NS * ROWS_OUT)])

    plsc.subcore_barrier()

    def _block(blk, _):
        r0 = s * ROWS_PER_SUB + blk * 2
        pltpu.sync_copy(meta_hbm.at[pl.ds(r0 * 8, 16)], meta_v)
        cps = [pltpu.async_copy(ntab_hbm.at[meta_v.at[8 * k]],
                                rows_v.at[pl.ds(k * 128, 128)], sm)
               for k, sm in ((0, sem), (1, sem2))]

        def _group(g, _):
            k = g >> 3
            l0 = (g & 7) * 16
            tvec = meta_v[8 * k + 2, pl.ds(l0, 16)]
            nvec = lax.bitcast_convert_type(meta_v[8 * k + 3, pl.ds(l0, 16)],
                                            jnp.float32)
            mvec = meta_v[8 * k + 4, pl.ds(l0, 16)]
            scale = jnp.where(mvec == c, nvec, 0.0)
            for e in range(16):
                tspl = _vbroadcast(tvec, e)
                sspl = _vbroadcast(scale, e)
                base = tspl * D + lax.iota(jnp.int32, 16)
                row = g * 16 + e
                for j in range(D // 16):
                    rr = plsc.load_gather(rtab_v, [base + j * 16])
                    rows_v[row, pl.ds(j * 16, 16)] = (
                        rows_v[row, pl.ds(j * 16, 16)] * rr * sspl)
            return 0

        for k in range(2):
            cps[k].wait()
            lax.fori_loop(8 * k, 8 * (k + 1), _group, 0)
            pltpu.sync_copy(rows_v.at[pl.ds(k * 128, 128)],
                            acc_sh.at[meta_v.at[8 * k + 1]], add=True)
        return 0

    lax.fori_loop(0, BLOCKS, _block, 0)
    plsc.subcore_barrier()
    pltpu.sync_copy(acc_sh.at[pl.ds(s * ROWS_OUT, ROWS_OUT)],
                    out_hbm.at[c, pl.ds(s * ROWS_OUT, ROWS_OUT)])

    @pl.when(s == NS - 1)
    def _out_tail():
        pltpu.sync_copy(acc_sh.at[pl.ds(NS * ROWS_OUT, N_ENT - NS * ROWS_OUT)],
                        out_hbm.at[c, pl.ds(NS * ROWS_OUT,
                                            N_ENT - NS * ROWS_OUT)])


def _mm_t(a, b):
    return lax.dot_general(a, b, (((1,), (1,)), ((), ())),
                           preferred_element_type=jnp.float32)


def _k1_body(acc_ref, nemb_ref, remb_ref, loop_ref, wi_ref, wo_ref, ws_ref,
             wr_ref, sub_ref, rel_ref, nf_ref, img_ref):
    comp_edge = _mm_t(acc_ref[0], wi_ref[...]) + _mm_t(acc_ref[1], wo_ref[...])
    pre = (_mm_t(nemb_ref[...] * loop_ref[...][0][None, :], ws_ref[...])
           + comp_edge) * (1.0 / 3.0)
    m = jnp.mean(pre, axis=0)
    cen = pre - m[None, :]
    v = jnp.mean(cen * cen, axis=0)
    nf = jnp.tanh(cen * lax.rsqrt(v + 1e-5)[None, :])
    nf_ref[...] = nf
    r_feats = _mm_t(remb_ref[...], wr_ref[...])
    sub_c = sub_ref[...]
    CH = 1000
    acc = jnp.zeros((B, D), jnp.float32)
    for k in range(N_ENT // CH):
        it = lax.broadcasted_iota(jnp.int32, (1, CH), 1) + (k * CH)
        oh = jnp.where(sub_c == it, 1.0, 0.0)
        acc = acc + jnp.dot(oh, nf[k * CH:(k + 1) * CH],
                            preferred_element_type=jnp.float32)
    sub_e = acc
    rel_c = rel_ref[...]
    itr = lax.broadcasted_iota(jnp.int32, (1, N_REL), 1)
    ohr = jnp.where(rel_c == itr, 1.0, 0.0)
    rel_e = jnp.dot(ohr, r_feats, preferred_element_type=jnp.float32)
    u_i = lax.broadcasted_iota(jnp.int32, (D, 2 * D), 1)
    d_i = lax.broadcasted_iota(jnp.int32, (D, 2 * D), 0)
    pe = jnp.where(u_i == 2 * d_i, 1.0, 0.0)
    po = jnp.where(u_i == 2 * d_i + 1, 1.0, 0.0)
    img = (jnp.dot(sub_e, pe, preferred_element_type=jnp.float32)
           + jnp.dot(rel_e, po, preferred_element_type=jnp.float32))
    m0 = jnp.mean(img)
    cen0 = img - m0
    v0 = jnp.mean(cen0 * cen0)
    img_ref[...] = cen0 * lax.rsqrt(v0 + 1e-5)


K2_GRID = 3
FPG = NFILT // K2_GRID
CW = FPG * OW * OH


def _k2_body(img_ref, t2_ref, fcw_ref, fcb_ref, x_ref, acc_ref):
    g = pl.program_id(0)
    y = jnp.dot(img_ref[...], t2_ref[...], preferred_element_type=jnp.float32)
    f_i = lax.broadcasted_iota(jnp.int32, (FPG, CW), 0)
    c_i = lax.broadcasted_iota(jnp.int32, (FPG, CW), 1)
    R = jnp.where(f_i == c_i // (OW * OH), 1.0, 0.0)
    colmean = jnp.mean(y, axis=0)[None, :]
    m1 = _mm_t(colmean, R) * (1.0 / (OW * OH))
    m1c = jnp.dot(m1, R, preferred_element_type=jnp.float32)
    yc = y - m1c
    colsq = jnp.mean(yc * yc, axis=0)[None, :]
    v1 = _mm_t(colsq, R) * (1.0 / (OW * OH))
    s1c = jnp.dot(lax.rsqrt(v1 + 1e-5), R, preferred_element_type=jnp.float32)
    y = jnp.maximum(yc * s1c, 0.0)
    part = _mm_t(y, fcw_ref[...])

    @pl.when(g == 0)
    def _init():
        acc_ref[...] = part

    @pl.when(g > 0)
    def _accum():
        acc_ref[...] = acc_ref[...] + part

    @pl.when(g == K2_GRID - 1)
    def _finish():
        x = acc_ref[...] + fcb_ref[...]
        m2 = jnp.mean(x, axis=0)
        cen2 = x - m2[None, :]
        v2 = jnp.mean(cen2 * cen2, axis=0)
        x_ref[...] = jnp.maximum(cen2 * lax.rsqrt(v2 + 1e-5)[None, :], 0.0)


def _k3_body(x_ref, nf_ref, out_ref):
    out_ref[...] = _mm_t(x_ref[...], nf_ref[...])


def _build_sel() -> np.ndarray:
    s = np.zeros((KER * KER, 2 * D, OW * OH), np.float32)
    for p in range(OW):
        for q in range(OH):
            for i in range(KER):
                for j in range(KER):
                    s[i * KER + j, (p + i) * 16 + (q + j), p * OH + q] = 1.0
    return s.reshape(KER * KER, 2 * D * OW * OH)


_SEL = _build_sel()


def _conv_toeplitz(conv_w):
    w49 = conv_w[:, 0].reshape(NFILT, KER * KER)
    t = jnp.dot(w49, jnp.asarray(_SEL))
    return t.reshape(NFILT, 2 * D, OW * OH).transpose(1, 0, 2).reshape(
        2 * D, FLAT)


def kernel(edge_index, etype, norm, in_edges_mask, out_edges_mask, sub, rel,
           n_embds, rel_embds, loop_rel, W_O, W_I, W_S, W_R, conv_w, fc_w,
           fc_b):
    src = edge_index[0]
    dst = edge_index[1]
    mcode = out_edges_mask.astype(jnp.int32)
    nflat = norm[:, 0]
    pad = EP - E
    src_p = jnp.pad(src, (0, pad)).reshape(EP // 128, 128)
    dst_p = jnp.pad(dst, (0, pad)).reshape(EP // 128, 128)
    et_p = jnp.pad(etype, (0, pad)).reshape(EP // 128, 128)
    nm_p = jnp.pad(nflat, (0, pad)).reshape(EP // 128, 128)
    nm_i = lax.bitcast_convert_type(nm_p, jnp.int32)
    mc_p = jnp.pad(mcode, (0, pad)).reshape(EP // 128, 128)
    zp = jnp.zeros_like(src_p)
    meta = jnp.stack([src_p, dst_p, et_p, nm_i, mc_p, zp, zp, zp],
                     axis=1).reshape(8 * EP // 128, 128)
    rtab = jnp.concatenate([rel_embds, loop_rel], axis=0).reshape(-1)

    acc = _sc_edge(meta, n_embds, rtab)

    nf, img = pl.pallas_call(
        _k1_body,
        out_shape=(
            jax.ShapeDtypeStruct((N_ENT, D), jnp.float32),
            jax.ShapeDtypeStruct((B, 2 * D), jnp.float32),
        ),
    )(acc, n_embds, rel_embds, loop_rel, W_I, W_O, W_S, W_R,
      sub.reshape(B, 1), rel.reshape(B, 1))

    t2 = _conv_toeplitz(conv_w)
    x = pl.pallas_call(
        _k2_body,
        grid=(K2_GRID,),
        in_specs=[
            pl.BlockSpec((B, 2 * D), lambda g: (0, 0)),
            pl.BlockSpec((2 * D, CW), lambda g: (0, g)),
            pl.BlockSpec((D, CW), lambda g: (0, g)),
            pl.BlockSpec((1, D), lambda g: (0, 0)),
        ],
        out_specs=pl.BlockSpec((B, D), lambda g: (0, 0)),
        scratch_shapes=[pltpu.VMEM((B, D), jnp.float32)],
        out_shape=jax.ShapeDtypeStruct((B, D), jnp.float32),
    )(img, t2, fc_w, fc_b.reshape(1, D))

    scores = pl.pallas_call(
        _k3_body,
        out_shape=jax.ShapeDtypeStruct((B, N_ENT), jnp.float32),
    )(x, nf)
    return scores


# speedup vs baseline: 1.8994x; 1.0004x over previous
import functools

import numpy as np
import jax
import jax.numpy as jnp
from jax import lax
from jax.experimental import pallas as pl
from jax.experimental.pallas import tpu as pltpu
from jax.experimental.pallas import tpu_sc as plsc

N_ENT = 10000
N_REL = 64
D = 128
E = 320000
B = 1024
KER = 7
NFILT = 96
OW = 10
OH = 10
FLAT = NFILT * OW * OH

NC = 2
NS = 16
EP = 327680
ROWS_PER_SUB = EP // 128 // NS
BLOCKS = ROWS_PER_SUB // 2
ROWS_OUT = 624

def _vbroadcast(vec, e):
    idx = jnp.full((16, 1), e, jnp.int32)
    dnums = lax.GatherDimensionNumbers(
        offset_dims=(), collapsed_slice_dims=(0,), start_index_map=(0,))
    return lax.gather(vec, idx, dnums, (1,),
                      mode=lax.GatherScatterMode.PROMISE_IN_BOUNDS)


_sc_mesh = plsc.VectorSubcoreMesh(core_axis_name="c", subcore_axis_name="s")


@functools.partial(
    pl.kernel,
    mesh=_sc_mesh,
    compiler_params=pltpu.CompilerParams(needs_layout_passes=False),
    out_type=jax.ShapeDtypeStruct((NC, N_ENT, D), jnp.float32),
    scratch_types=[
        pltpu.VMEM((16, 128), jnp.int32),
        pltpu.VMEM((256, D), jnp.float32),
        pltpu.VMEM(((N_REL + 1) * D,), jnp.float32),
        pltpu.VMEM_SHARED((N_ENT, D), jnp.float32),
        pltpu.SemaphoreType.DMA,
        pltpu.SemaphoreType.DMA,
    ],
)
def _sc_edge(meta_hbm, ntab_hbm, rtab_hbm,
             out_hbm, meta_v, rows_v, rtab_v,
             acc_sh, sem, sem2):
    c = lax.axis_index("c")
    s = lax.axis_index("s")

    pltpu.sync_copy(rtab_hbm, rtab_v)

    def _zero_row(i, _):
        for j in range(D // 16):
            rows_v[i, pl.ds(j * 16, 16)] = jnp.zeros((16,), jnp.float32)
        return 0

    lax.fori_loop(0, 256, _zero_row, 0)
    pltpu.sync_copy(rows_v, acc_sh.at[pl.ds(s * ROWS_OUT, 256)])
    pltpu.sync_copy(rows_v, acc_sh.at[pl.ds(s * ROWS_OUT + 256, 256)])
    pltpu.sync_copy(rows_v.at[pl.ds(0, ROWS_OUT - 512)],
                    acc_sh.at[pl.ds(s * ROWS_OUT + 512, ROWS_OUT - 512)])

    @pl.when(s == NS - 1)
    def _zero_tail():
        pltpu.sync_copy(rows_v.at[pl.ds(0, N_ENT - NS * ROWS_OUT)],
                        acc_sh.at[pl.ds(NS * ROWS_OUT, N_ENT - NS * ROWS_OUT)])

    plsc.subcore_barrier()

    def _block(blk, _):
        r0 = s * ROWS_PER_SUB + blk * 2
        pltpu.sync_copy(meta_hbm.at[pl.ds(r0 * 8, 16)], meta_v)
        cps = [pltpu.async_copy(ntab_hbm.at[meta_v.at[8 * k]],
                                rows_v.at[pl.ds(k * 128, 128)], sm)
               for k, sm in ((0, sem), (1, sem2))]

        def _group(g, _):
            k = g >> 3
            l0 = (g & 7) * 16
            tvec = meta_v[8 * k + 2, pl.ds(l0, 16)]
            nvec = lax.bitcast_convert_type(meta_v[8 * k + 3, pl.ds(l0, 16)],
                                            jnp.float32)
            mvec = meta_v[8 * k + 4, pl.ds(l0, 16)]
            scale = jnp.where(mvec == c, nvec, 0.0)
            for e in range(16):
                tspl = _vbroadcast(tvec, e)
                sspl = _vbroadcast(scale, e)
                base = tspl + lax.iota(jnp.int32, 16)
                row = g * 16 + e
                for j in range(D // 16):
                    rr = plsc.load_gather(rtab_v, [base + j * 16])
                    rows_v[row, pl.ds(j * 16, 16)] = (
                        rows_v[row, pl.ds(j * 16, 16)] * rr * sspl)
            return 0

        for k in range(2):
            cps[k].wait()
            lax.fori_loop(8 * k, 8 * (k + 1), _group, 0)
            pltpu.sync_copy(rows_v.at[pl.ds(k * 128, 128)],
                            acc_sh.at[meta_v.at[8 * k + 1]], add=True)
        return 0

    lax.fori_loop(0, BLOCKS, _block, 0)
    plsc.subcore_barrier()
    pltpu.sync_copy(acc_sh.at[pl.ds(s * ROWS_OUT, ROWS_OUT)],
                    out_hbm.at[c, pl.ds(s * ROWS_OUT, ROWS_OUT)])

    @pl.when(s == NS - 1)
    def _out_tail():
        pltpu.sync_copy(acc_sh.at[pl.ds(NS * ROWS_OUT, N_ENT - NS * ROWS_OUT)],
                        out_hbm.at[c, pl.ds(NS * ROWS_OUT,
                                            N_ENT - NS * ROWS_OUT)])


def _mm_t(a, b):
    return lax.dot_general(a, b, (((1,), (1,)), ((), ())),
                           preferred_element_type=jnp.float32)


def _k1_body(acc_ref, nemb_ref, remb_ref, loop_ref, wi_ref, wo_ref, ws_ref,
             wr_ref, sub_ref, rel_ref, nf_ref, img_ref):
    comp_edge = _mm_t(acc_ref[0], wi_ref[...]) + _mm_t(acc_ref[1], wo_ref[...])
    pre = (_mm_t(nemb_ref[...] * loop_ref[...][0][None, :], ws_ref[...])
           + comp_edge) * (1.0 / 3.0)
    m = jnp.mean(pre, axis=0)
    cen = pre - m[None, :]
    v = jnp.mean(cen * cen, axis=0)
    nf = jnp.tanh(cen * lax.rsqrt(v + 1e-5)[None, :])
    nf_ref[...] = nf
    r_feats = _mm_t(remb_ref[...], wr_ref[...])
    sub_c = sub_ref[...]
    CH = 1000
    acc = jnp.zeros((B, D), jnp.float32)
    for k in range(N_ENT // CH):
        it = lax.broadcasted_iota(jnp.int32, (1, CH), 1) + (k * CH)
        oh = jnp.where(sub_c == it, 1.0, 0.0)
        acc = acc + jnp.dot(oh, nf[k * CH:(k + 1) * CH],
                            preferred_element_type=jnp.float32)
    sub_e = acc
    rel_c = rel_ref[...]
    itr = lax.broadcasted_iota(jnp.int32, (1, N_REL), 1)
    ohr = jnp.where(rel_c == itr, 1.0, 0.0)
    rel_e = jnp.dot(ohr, r_feats, preferred_element_type=jnp.float32)
    u_i = lax.broadcasted_iota(jnp.int32, (D, 2 * D), 1)
    d_i = lax.broadcasted_iota(jnp.int32, (D, 2 * D), 0)
    pe = jnp.where(u_i == 2 * d_i, 1.0, 0.0)
    po = jnp.where(u_i == 2 * d_i + 1, 1.0, 0.0)
    img = (jnp.dot(sub_e, pe, preferred_element_type=jnp.float32)
           + jnp.dot(rel_e, po, preferred_element_type=jnp.float32))
    m0 = jnp.mean(img)
    cen0 = img - m0
    v0 = jnp.mean(cen0 * cen0)
    img_ref[...] = cen0 * lax.rsqrt(v0 + 1e-5)


K2_GRID = 3
FPG = NFILT // K2_GRID
CW = FPG * OW * OH


def _k2_body(img_ref, t2_ref, fcw_ref, fcb_ref, x_ref, acc_ref):
    g = pl.program_id(0)
    y = jnp.dot(img_ref[...], t2_ref[...], preferred_element_type=jnp.float32)
    f_i = lax.broadcasted_iota(jnp.int32, (FPG, CW), 0)
    c_i = lax.broadcasted_iota(jnp.int32, (FPG, CW), 1)
    R = jnp.where(f_i == c_i // (OW * OH), 1.0, 0.0)
    colmean = jnp.mean(y, axis=0)[None, :]
    m1 = _mm_t(colmean, R) * (1.0 / (OW * OH))
    m1c = jnp.dot(m1, R, preferred_element_type=jnp.float32)
    yc = y - m1c
    colsq = jnp.mean(yc * yc, axis=0)[None, :]
    v1 = _mm_t(colsq, R) * (1.0 / (OW * OH))
    s1c = jnp.dot(lax.rsqrt(v1 + 1e-5), R, preferred_element_type=jnp.float32)
    y = jnp.maximum(yc * s1c, 0.0)
    part = _mm_t(y, fcw_ref[...])

    @pl.when(g == 0)
    def _init():
        acc_ref[...] = part

    @pl.when(g > 0)
    def _accum():
        acc_ref[...] = acc_ref[...] + part

    @pl.when(g == K2_GRID - 1)
    def _finish():
        x = acc_ref[...] + fcb_ref[...]
        m2 = jnp.mean(x, axis=0)
        cen2 = x - m2[None, :]
        v2 = jnp.mean(cen2 * cen2, axis=0)
        x_ref[...] = jnp.maximum(cen2 * lax.rsqrt(v2 + 1e-5)[None, :], 0.0)


def _k3_body(x_ref, nf_ref, out_ref):
    out_ref[...] = _mm_t(x_ref[...], nf_ref[...])


def _build_sel() -> np.ndarray:
    s = np.zeros((KER * KER, 2 * D, OW * OH), np.float32)
    for p in range(OW):
        for q in range(OH):
            for i in range(KER):
                for j in range(KER):
                    s[i * KER + j, (p + i) * 16 + (q + j), p * OH + q] = 1.0
    return s.reshape(KER * KER, 2 * D * OW * OH)


_SEL = _build_sel()


def _conv_toeplitz(conv_w):
    w49 = conv_w[:, 0].reshape(NFILT, KER * KER)
    t = jnp.dot(w49, jnp.asarray(_SEL))
    return t.reshape(NFILT, 2 * D, OW * OH).transpose(1, 0, 2).reshape(
        2 * D, FLAT)


def kernel(edge_index, etype, norm, in_edges_mask, out_edges_mask, sub, rel,
           n_embds, rel_embds, loop_rel, W_O, W_I, W_S, W_R, conv_w, fc_w,
           fc_b):
    src = edge_index[0]
    dst = edge_index[1]
    mcode = out_edges_mask.astype(jnp.int32)
    nflat = norm[:, 0]
    pad = EP - E
    src_p = jnp.pad(src, (0, pad)).reshape(EP // 128, 128)
    dst_p = jnp.pad(dst, (0, pad)).reshape(EP // 128, 128)
    et_p = (jnp.pad(etype, (0, pad)) * D).reshape(EP // 128, 128)
    nm_p = jnp.pad(nflat, (0, pad)).reshape(EP // 128, 128)
    nm_i = lax.bitcast_convert_type(nm_p, jnp.int32)
    mc_p = jnp.pad(mcode, (0, pad)).reshape(EP // 128, 128)
    zp = jnp.zeros_like(src_p)
    meta = jnp.stack([src_p, dst_p, et_p, nm_i, mc_p, zp, zp, zp],
                     axis=1).reshape(8 * EP // 128, 128)
    rtab = jnp.concatenate([rel_embds, loop_rel], axis=0).reshape(-1)

    acc = _sc_edge(meta, n_embds, rtab)

    nf, img = pl.pallas_call(
        _k1_body,
        out_shape=(
            jax.ShapeDtypeStruct((N_ENT, D), jnp.float32),
            jax.ShapeDtypeStruct((B, 2 * D), jnp.float32),
        ),
    )(acc, n_embds, rel_embds, loop_rel, W_I, W_O, W_S, W_R,
      sub.reshape(B, 1), rel.reshape(B, 1))

    t2 = _conv_toeplitz(conv_w)
    x = pl.pallas_call(
        _k2_body,
        grid=(K2_GRID,),
        in_specs=[
            pl.BlockSpec((B, 2 * D), lambda g: (0, 0)),
            pl.BlockSpec((2 * D, CW), lambda g: (0, g)),
            pl.BlockSpec((D, CW), lambda g: (0, g)),
            pl.BlockSpec((1, D), lambda g: (0, 0)),
        ],
        out_specs=pl.BlockSpec((B, D), lambda g: (0, 0)),
        scratch_shapes=[pltpu.VMEM((B, D), jnp.float32)],
        out_shape=jax.ShapeDtypeStruct((B, D), jnp.float32),
    )(img, t2, fc_w, fc_b.reshape(1, D))

    scores = pl.pallas_call(
        _k3_body,
        out_shape=jax.ShapeDtypeStruct((B, N_ENT), jnp.float32),
    )(x, nf)
    return scores
